# baseline (speedup 1.0000x reference)
import functools

import jax
import jax.numpy as jnp
from jax import lax
from jax.experimental import pallas as pl
from jax.experimental.pallas import tpu as pltpu
from jax.experimental.pallas import tpu_sc as plsc

_BATCH = 1024
_FEAT = 64
_NUM_CLASSES = 100000
_LANES = 16

_NC = 2
_NS = 16
_NW = _NC * _NS
_BPW = _BATCH // _NW
_TW = 128
_GRP = 4
_TAIL0 = (_NUM_CLASSES // _TW) * _TW
_LASTW = ((_NUM_CLASSES - _TW) // _TW)


@functools.partial(
    pl.kernel,
    mesh=plsc.VectorSubcoreMesh(core_axis_name="c", subcore_axis_name="s"),
    compiler_params=pltpu.CompilerParams(
        needs_layout_passes=False, skip_device_barrier=True),
    out_type=jax.ShapeDtypeStruct((_NW, _LANES), jnp.float32),
    scratch_types=[
        pltpu.VMEM((_BPW,), jnp.int32),
        pltpu.VMEM((_BPW, _FEAT), jnp.float32),
        pltpu.VMEM((3 * _GRP, _FEAT, _TW), jnp.float32),
        pltpu.VMEM((_FEAT, _TW), jnp.float32),
        pltpu.VMEM((_LANES,), jnp.float32),
        pltpu.SemaphoreType.DMA,
        pltpu.SemaphoreType.DMA,
        pltpu.SemaphoreType.DMA,
        pltpu.SemaphoreType.DMA,
    ],
)
def _center_loss_partials(feat_hbm, tgt_hbm, ct_hbm, tail_hbm, out_hbm,
                          idx_v, f_v, slab_v, tail_v, o_v, fsem,
                          gsem_a, gsem_b, gsem_c):
    wid = lax.axis_index("s") * _NC + lax.axis_index("c")
    base = wid * _BPW
    pltpu.sync_copy(tgt_hbm.at[pl.ds(base, _BPW)], idx_v)
    pltpu.sync_copy(tail_hbm, tail_v)
    fcp = pltpu.async_copy(feat_hbm.at[pl.ds(base, _BPW)], f_v, fsem)

    tgt_rows = [None] * _BPW
    for g in range(_BPW // _LANES):
        tv = idx_v[pl.ds(g * _LANES, _LANES)]
        for i in range(_LANES):
            tgt_rows[g * _LANES + i] = tv[i]

    sems = [gsem_a, gsem_b, gsem_c]
    nbatch = _BPW // _GRP
    _D = 3

    def fire_batch(b):
        cps = []
        for k in range(_GRP):
            r = tgt_rows[b * _GRP + k]
            col0 = pl.multiple_of(
                jnp.minimum(r // _TW, jnp.int32(_LASTW)) * _TW, _TW)
            cps.append(
                pltpu.async_copy(ct_hbm.at[:, pl.ds(col0, _TW)],
                                 slab_v.at[(b % _D) * _GRP + k],
                                 sems[b % _D]))
        return cps

    lanes = lax.iota(jnp.int32, _LANES)
    fcp.wait()
    inflight = [fire_batch(0), fire_batch(1)]
    acc = jnp.zeros((_LANES,), jnp.float32)
    for b in range(nbatch):
        if b + 2 < nbatch:
            inflight.append(fire_batch(b + 2))
        for cp in inflight.pop(0):
            cp.wait()
        for k in range(_GRP):
            i = b * _GRP + k
            r = tgt_rows[i]
            cloc = jnp.full((_LANES,), r % _TW, jnp.int32)
            in_tail = r >= jnp.int32(_TAIL0)
            sbuf = slab_v.at[(b % _D) * _GRP + k]
            for ch in range(_FEAT // _LANES):
                dims = lanes + jnp.int32(ch * _LANES)
                cvals = jnp.where(in_tail,
                                  plsc.load_gather(tail_v, [dims, cloc]),
                                  plsc.load_gather(sbuf, [dims, cloc]))
                df = f_v[i, pl.ds(ch * _LANES, _LANES)] - cvals
                acc = acc + df * df
    o_v[...] = acc
    pltpu.sync_copy(o_v, out_hbm.at[wid])


def kernel(features, target, centers):
    ct = centers.T
    tail = jnp.pad(ct[:, _TAIL0:], ((0, 0), (0, _TW - (_NUM_CLASSES - _TAIL0))))
    partials = _center_loss_partials(features, target, ct, tail)
    zero_term = jnp.float32((_NUM_CLASSES - 1) * 1e-12)
    return jnp.sum(partials) / jnp.float32(_BATCH) + zero_term


# speedup vs baseline: 1.0584x; 1.0584x over previous
import functools

import jax
import jax.numpy as jnp
from jax import lax
from jax.experimental import pallas as pl
from jax.experimental.pallas import tpu as pltpu
from jax.experimental.pallas import tpu_sc as plsc

_BATCH = 1024
_FEAT = 64
_NUM_CLASSES = 100000
_LANES = 16

_NC = 2
_NS = 16
_NW = _NC * _NS
_BPW = _BATCH // _NW
_TW = 128
_GRP = 4
_TAIL0 = (_NUM_CLASSES // _TW) * _TW
_LASTW = ((_NUM_CLASSES - _TW) // _TW)


@functools.partial(
    pl.kernel,
    mesh=plsc.VectorSubcoreMesh(core_axis_name="c", subcore_axis_name="s"),
    compiler_params=pltpu.CompilerParams(
        needs_layout_passes=False, skip_device_barrier=True),
    out_type=jax.ShapeDtypeStruct((_NW, _LANES), jnp.float32),
    scratch_types=[
        pltpu.VMEM((_BPW,), jnp.int32),
        pltpu.VMEM((_BPW, _FEAT), jnp.float32),
        pltpu.VMEM((3 * _GRP, _FEAT, _TW), jnp.float32),
        pltpu.VMEM((_LANES,), jnp.float32),
        pltpu.SemaphoreType.DMA,
        pltpu.SemaphoreType.DMA,
        pltpu.SemaphoreType.DMA,
        pltpu.SemaphoreType.DMA,
    ],
)
def _center_loss_partials(feat_hbm, tgt_hbm, ct_hbm, tail_hbm, out_hbm,
                          idx_v, f_v, slab_v, o_v, fsem,
                          gsem_a, gsem_b, gsem_c):
    wid = lax.axis_index("s") * _NC + lax.axis_index("c")
    base = wid * _BPW
    pltpu.sync_copy(tgt_hbm.at[pl.ds(base, _BPW)], idx_v)
    fcp = pltpu.async_copy(feat_hbm.at[pl.ds(base, _BPW)], f_v, fsem)

    tgt_rows = [None] * _BPW
    for g in range(_BPW // _LANES):
        tv = idx_v[pl.ds(g * _LANES, _LANES)]
        for i in range(_LANES):
            tgt_rows[g * _LANES + i] = tv[i]

    sems = [gsem_a, gsem_b, gsem_c]
    nbatch = _BPW // _GRP
    _D = 3

    def fire_batch(b):
        for k in range(_GRP):
            r = tgt_rows[b * _GRP + k]
            col0 = pl.multiple_of(
                jnp.minimum(r // _TW, jnp.int32(_LASTW)) * _TW, _TW)
            sbuf = slab_v.at[(b % _D) * _GRP + k]
            sem = sems[b % _D]

            @pl.when(r < jnp.int32(_TAIL0))
            def _fire_window():
                pltpu.async_copy(ct_hbm.at[:, pl.ds(col0, _TW)], sbuf, sem)

            @pl.when(r >= jnp.int32(_TAIL0))
            def _fire_tail():
                pltpu.async_copy(tail_hbm, sbuf, sem)

    def drain_batch(b):
        for k in range(_GRP):
            pltpu.make_async_copy(ct_hbm.at[:, pl.ds(0, _TW)],
                                  slab_v.at[(b % _D) * _GRP + k],
                                  sems[b % _D]).wait()

    lanes = lax.iota(jnp.int32, _LANES)
    fcp.wait()
    fire_batch(0)
    fire_batch(1)
    acc = jnp.zeros((_LANES,), jnp.float32)
    for b in range(nbatch):
        if b + 2 < nbatch:
            fire_batch(b + 2)
        drain_batch(b)
        for k in range(_GRP):
            i = b * _GRP + k
            r = tgt_rows[i]
            cloc = jnp.full((_LANES,), r % _TW, jnp.int32)
            sbuf = slab_v.at[(b % _D) * _GRP + k]
            for ch in range(_FEAT // _LANES):
                dims = lanes + jnp.int32(ch * _LANES)
                cvals = plsc.load_gather(sbuf, [dims, cloc])
                df = f_v[i, pl.ds(ch * _LANES, _LANES)] - cvals
                acc = acc + df * df
    o_v[...] = acc
    pltpu.sync_copy(o_v, out_hbm.at[wid])


def kernel(features, target, centers):
    ct = centers.T
    tail = jnp.pad(ct[:, _TAIL0:], ((0, 0), (0, _TW - (_NUM_CLASSES - _TAIL0))))
    partials = _center_loss_partials(features, target, ct, tail)
    zero_term = jnp.float32((_NUM_CLASSES - 1) * 1e-12)
    return jnp.sum(partials) / jnp.float32(_BATCH) + zero_term
